# baseline (speedup 1.0000x reference)
import functools

import jax
import jax.numpy as jnp
from jax import lax
from jax.experimental import pallas as pl
from jax.experimental.pallas import tpu as pltpu
from jax.experimental.pallas import tpu_sc as plsc

NUM_CORES = 2
NUM_SUBCORES = 16
NUM_WORKERS = NUM_CORES * NUM_SUBCORES
BATCH = 16384
EMBED_DIM = 6
VOCAB_ROWS = 1000000
B_PER_W = BATCH // NUM_WORKERS
CHUNK = 128
N_CHUNKS = B_PER_W // CHUNK

_mesh = plsc.VectorSubcoreMesh(
    core_axis_name="c", subcore_axis_name="s",
    num_cores=NUM_CORES, num_subcores=NUM_SUBCORES,
)


@functools.partial(
    pl.kernel,
    out_type=jax.ShapeDtypeStruct((EMBED_DIM, BATCH), jnp.float32),
    mesh=_mesh,
    compiler_params=pltpu.CompilerParams(
        use_tc_tiling_on_sc=False, needs_layout_passes=False),
    scratch_types=[
        pltpu.VMEM((B_PER_W,), jnp.int32),
        pltpu.VMEM((EMBED_DIM, B_PER_W), jnp.float32),
        pltpu.SemaphoreType.DMA,
    ],
)
def _sc_gather(idx_hbm, tab_t, out_t, idx_v, cols_v, sem):
    wid = lax.axis_index("s") * NUM_CORES + lax.axis_index("c")
    base = wid * B_PER_W
    pltpu.sync_copy(idx_hbm.at[pl.ds(base, B_PER_W)], idx_v)
    copies = []
    for k in range(EMBED_DIM):
        plane = tab_t.at[k]
        for j in range(N_CHUNKS):
            copies.append(
                pltpu.async_copy(
                    plane.at[idx_v.at[pl.ds(j * CHUNK, CHUNK)]],
                    cols_v.at[k, pl.ds(j * CHUNK, CHUNK)],
                    sem,
                )
            )
    for c in copies:
        c.wait()
    for k in range(EMBED_DIM):
        pltpu.sync_copy(cols_v.at[k], out_t.at[k, pl.ds(base, B_PER_W)])


def kernel(indices, table):
    idx = indices.astype(jnp.int32)
    return _sc_gather(idx, table.T).T


# speedup vs baseline: 3.8141x; 3.8141x over previous
import functools

import jax
import jax.numpy as jnp
from jax import lax
from jax.experimental import pallas as pl
from jax.experimental.pallas import tpu as pltpu
from jax.experimental.pallas import tpu_sc as plsc

NUM_CORES = 2
NUM_SUBCORES = 16
NUM_WORKERS = NUM_CORES * NUM_SUBCORES
BATCH = 16384
EMBED_DIM = 6
VOCAB_ROWS = 1000000
B_PER_W = BATCH // NUM_WORKERS
CHUNK = 128
N_CHUNKS = B_PER_W // CHUNK

_mesh = plsc.VectorSubcoreMesh(
    core_axis_name="c", subcore_axis_name="s",
    num_cores=NUM_CORES, num_subcores=NUM_SUBCORES,
)


@functools.partial(
    pl.kernel,
    out_type=jax.ShapeDtypeStruct((EMBED_DIM, BATCH), jnp.float32),
    mesh=_mesh,
    compiler_params=pltpu.CompilerParams(
        use_tc_tiling_on_sc=False, needs_layout_passes=False),
    scratch_types=[
        pltpu.VMEM((B_PER_W,), jnp.int32),
        pltpu.VMEM((EMBED_DIM, B_PER_W), jnp.float32),
        pltpu.SemaphoreType.DMA,
    ],
)
def _sc_gather(idx_hbm, p0, p1, p2, p3, p4, p5, out_t, idx_v, cols_v, sem):
    wid = lax.axis_index("s") * NUM_CORES + lax.axis_index("c")
    base = wid * B_PER_W
    pltpu.sync_copy(idx_hbm.at[pl.ds(base, B_PER_W)], idx_v)
    copies = []
    planes = (p0, p1, p2, p3, p4, p5)
    for k in range(EMBED_DIM):
        plane = planes[k]
        for j in range(N_CHUNKS):
            copies.append(
                pltpu.async_copy(
                    plane.at[idx_v.at[pl.ds(j * CHUNK, CHUNK)]],
                    cols_v.at[k, pl.ds(j * CHUNK, CHUNK)],
                    sem,
                )
            )
    for c in copies:
        c.wait()
    for k in range(EMBED_DIM):
        pltpu.sync_copy(cols_v.at[k], out_t.at[k, pl.ds(base, B_PER_W)])


def kernel(indices, table):
    idx = indices.astype(jnp.int32)
    cols = [table[:, k] for k in range(EMBED_DIM)]
    return _sc_gather(idx, *cols).T
